# baseline (speedup 1.0000x reference)
import functools

import jax
import jax.numpy as jnp
import numpy as np
from jax import lax
from jax.experimental import pallas as pl
from jax.experimental.pallas import tpu as pltpu
from jax.experimental.pallas import tpu_sc as plsc

_N_FFT = 4410
_SAMPLING_RATE = 44100
_N_CHROMA = 12


def _semitone_bins() -> list[int]:
    freq = np.linspace(0.0, _SAMPLING_RATE / 2, _N_FFT // 2 + 1, dtype=np.float32)
    with np.errstate(divide="ignore", invalid="ignore"):
        mapping = (
            np.float32(_N_CHROMA) * np.log2(freq / np.float32(440.0))
        ) % np.float32(_N_CHROMA)
    return [int(i) for i in np.argwhere(mapping == np.float32(0.0)).ravel()]


_BINS = _semitone_bins()


@jax.jit
def kernel(X):
    B, C, F, T = X.shape
    K = len(_BINS)
    NW = 32
    CHUNKS = NW // B
    TW = T // CHUNKS
    L = 16

    mesh = plsc.VectorSubcoreMesh(core_axis_name="c", subcore_axis_name="s")

    @functools.partial(
        pl.kernel,
        out_type=jax.ShapeDtypeStruct((B, C, _N_CHROMA, T), jnp.float32),
        mesh=mesh,
        scratch_types=[
            pltpu.VMEM((K, 8, TW), jnp.float32),
            pltpu.VMEM((_N_CHROMA, TW), jnp.float32),
            pltpu.SemaphoreType.DMA,
        ],
        compiler_params=pltpu.CompilerParams(use_tc_tiling_on_sc=True),
    )
    def chroma_sc(x_hbm, out_hbm, rows_v, out_v, sem):
        wid = lax.axis_index("s") * 2 + lax.axis_index("c")
        b = wid // CHUNKS
        t0 = (wid % CHUNKS) * TW

        handles = [
            pltpu.make_async_copy(
                x_hbm.at[b, 0, pl.ds((_BINS[k] // 8) * 8, 8), pl.ds(t0, TW)],
                rows_v.at[k],
                sem,
            )
            for k in range(K)
        ]
        for h in handles:
            h.start()
        for h in handles:
            h.wait()

        for i in range(TW // L):
            s = pl.ds(i * L, L)
            acc = rows_v[0, _BINS[0] % 8, s]
            for k in range(1, K):
                acc = acc + rows_v[k, _BINS[k] % 8, s]
            for j in range(_N_CHROMA):
                out_v[j, s] = acc

        pltpu.sync_copy(out_v, out_hbm.at[b, 0, :, pl.ds(t0, TW)])

    return chroma_sc(X)


# speedup vs baseline: 5.3524x; 5.3524x over previous
import functools

import jax
import jax.numpy as jnp
import numpy as np
from jax import lax
from jax.experimental import pallas as pl
from jax.experimental.pallas import tpu as pltpu
from jax.experimental.pallas import tpu_sc as plsc

_N_FFT = 4410
_SAMPLING_RATE = 44100
_N_CHROMA = 12


def _semitone_bins() -> list[int]:
    freq = np.linspace(0.0, _SAMPLING_RATE / 2, _N_FFT // 2 + 1, dtype=np.float32)
    with np.errstate(divide="ignore", invalid="ignore"):
        mapping = (
            np.float32(_N_CHROMA) * np.log2(freq / np.float32(440.0))
        ) % np.float32(_N_CHROMA)
    return [int(i) for i in np.argwhere(mapping == np.float32(0.0)).ravel()]


_BINS = _semitone_bins()


@jax.jit
def kernel(X):
    B, C, F, T = X.shape
    K = len(_BINS)
    NW = 32
    CHUNKS = NW // B
    TW = T // CHUNKS
    L = 16

    mesh = plsc.VectorSubcoreMesh(core_axis_name="c", subcore_axis_name="s")

    @functools.partial(
        pl.kernel,
        out_type=jax.ShapeDtypeStruct((B, C, _N_CHROMA, T), jnp.float32),
        mesh=mesh,
        scratch_types=[
            pltpu.VMEM((K, TW), jnp.float32),
            pltpu.VMEM((_N_CHROMA, TW), jnp.float32),
            pltpu.SemaphoreType.DMA,
        ],
        compiler_params=pltpu.CompilerParams(use_tc_tiling_on_sc=False),
    )
    def chroma_sc(x_hbm, out_hbm, rows_v, out_v, sem):
        wid = lax.axis_index("s") * 2 + lax.axis_index("c")
        b = wid // CHUNKS
        t0 = (wid % CHUNKS) * TW

        handles = [
            pltpu.make_async_copy(
                x_hbm.at[b, 0, _BINS[k], pl.ds(t0, TW)], rows_v.at[k], sem
            )
            for k in range(K)
        ]
        for h in handles:
            h.start()
        for h in handles:
            h.wait()

        for i in range(TW // L):
            s = pl.ds(i * L, L)
            acc = rows_v[0, s]
            for k in range(1, K):
                acc = acc + rows_v[k, s]
            for j in range(_N_CHROMA):
                out_v[j, s] = acc

        pltpu.sync_copy(out_v, out_hbm.at[b, 0, :, pl.ds(t0, TW)])

    return chroma_sc(X)
